# baseline (speedup 1.0000x reference)
import jax
import jax.numpy as jnp
from jax import lax
from jax.experimental import pallas as pl
from jax.experimental.pallas import tpu as pltpu
from jax.experimental.pallas import tpu_sc as plsc

NC = 2
NS = 16
NW = NC * NS


def _gather_body(idx_hbm, table_hbm, out_hbm, idx_v, rows_v, sem_g, sem_o,
                 sem_i):
    L, B = idx_hbm.shape
    D = table_hbm.shape[1]
    CB = B // NW
    wid = lax.axis_index("s") * NC + lax.axis_index("c")
    b0 = wid * CB

    def issue_gather(p, _):
        for j in range(4):
            sub = CB // 4
            pltpu.async_copy(
                table_hbm.at[idx_v.at[p].at[pl.ds(j * sub, sub)]],
                rows_v.at[p].at[pl.ds(j * sub, sub)],
                sem_g,
            )

    def drain_gather(p):
        pltpu.make_async_copy(
            table_hbm.at[pl.ds(0, CB)], rows_v.at[p], sem_g
        ).wait()

    def fetch_idx(l, p):
        return pltpu.async_copy(
            idx_hbm.at[lax.rem(l, L)].at[pl.ds(b0, CB)], idx_v.at[p], sem_i
        )

    def wait_idx(p):
        pltpu.make_async_copy(
            idx_hbm.at[0].at[pl.ds(0, CB)], idx_v.at[p], sem_i
        ).wait()

    def issue_write(l, p):
        return pltpu.async_copy(
            rows_v.at[p],
            out_hbm.at[pl.ds(b0, CB), pl.ds(l * D, D)],
            sem_o,
        )

    def wait_write(p):
        pltpu.make_async_copy(
            rows_v.at[p], out_hbm.at[pl.ds(0, CB), pl.ds(0, D)], sem_o
        ).wait()

    pltpu.sync_copy(idx_hbm.at[0].at[pl.ds(b0, CB)], idx_v.at[0])
    issue_gather(0, None)
    fetch_idx(1, 1)

    def body(l, carry):
        p = lax.rem(l, 2)
        q = 1 - p
        drain_gather(p)

        @pl.when(l > 0)
        def _():
            wait_write(q)

        issue_write(l, p)
        wait_idx(q)
        fetch_idx(l + 2, p)
        issue_gather(q, None)
        return carry

    lax.fori_loop(0, L - 1, body, 0)

    last = L - 1
    p = lax.rem(last, 2)
    drain_gather(p)
    wait_write(1 - p)
    issue_write(last, p)
    wait_idx(1 - p)
    wait_write(p)


def kernel(x, embedding):
    B, L = x.shape
    D = embedding.shape[1]
    assert B % NW == 0
    CB = B // NW
    idx_t = x.T.astype(jnp.int32)

    mesh = plsc.VectorSubcoreMesh(core_axis_name="c", subcore_axis_name="s")
    run = pl.kernel(
        _gather_body,
        out_type=jax.ShapeDtypeStruct((B, L * D), jnp.float32),
        mesh=mesh,
        scratch_types=[
            pltpu.VMEM((2, CB), jnp.int32),
            pltpu.VMEM((2, CB, D), jnp.float32),
            pltpu.SemaphoreType.DMA,
            pltpu.SemaphoreType.DMA,
            pltpu.SemaphoreType.DMA,
        ],
        compiler_params=pltpu.CompilerParams(use_tc_tiling_on_sc=False),
    )
    out = run(idx_t, embedding)
    return out.reshape(B, L, D)
